# baseline (device time: 4258637 ns/iter reference)
import jax
import jax.numpy as jnp
from jax import lax
from jax.experimental import pallas as pl
from jax.experimental.pallas import tpu as pltpu

C = 16


def kernel(x):
    m_per, n = x.shape
    half = m_per // 2
    rows = half // C

    def body(x_ref, out_ref, local_sem, sx, rx, sy, ry):
        mx = lax.axis_index("x")
        my = lax.axis_index("y")
        mz = lax.axis_index("z")
        partner_x = (1 - mx, my, mz)
        partner_y = (mx, 1 - my, mz)

        local = pltpu.make_async_copy(
            x_ref, out_ref.at[pl.ds(mx * m_per, m_per), :], local_sem
        )
        local.start()

        x_rdmas = []
        for i in range(C):
            src_off = my * half + i * rows
            dst_off = mx * m_per + my * half + i * rows
            r = pltpu.make_async_remote_copy(
                src_ref=x_ref.at[pl.ds(src_off, rows), :],
                dst_ref=out_ref.at[pl.ds(dst_off, rows), :],
                send_sem=sx.at[i],
                recv_sem=rx.at[i],
                device_id=partner_x,
                device_id_type=pl.DeviceIdType.MESH,
            )
            r.start()
            x_rdmas.append(r)

        y_rdmas = []
        for i in range(C):
            x_rdmas[i].wait_recv()
            off = (1 - mx) * m_per + my * half + i * rows
            r = pltpu.make_async_remote_copy(
                src_ref=out_ref.at[pl.ds(off, rows), :],
                dst_ref=out_ref.at[pl.ds(off, rows), :],
                send_sem=sy.at[i],
                recv_sem=ry.at[i],
                device_id=partner_y,
                device_id_type=pl.DeviceIdType.MESH,
            )
            r.start()
            y_rdmas.append(r)

        for i in range(C):
            x_rdmas[i].wait_send()
            y_rdmas[i].wait()
        local.wait()

    return pl.pallas_call(
        body,
        out_shape=jax.ShapeDtypeStruct((2 * m_per, n), x.dtype),
        in_specs=[pl.BlockSpec(memory_space=pltpu.MemorySpace.HBM)],
        out_specs=pl.BlockSpec(memory_space=pltpu.MemorySpace.HBM),
        scratch_shapes=[
            pltpu.SemaphoreType.DMA,
            pltpu.SemaphoreType.DMA((C,)),
            pltpu.SemaphoreType.DMA((C,)),
            pltpu.SemaphoreType.DMA((C,)),
            pltpu.SemaphoreType.DMA((C,)),
        ],
    )(x)


# device time: 989910 ns/iter; 4.3020x vs baseline; 4.3020x over previous
import jax
import jax.numpy as jnp
from jax import lax
from jax.experimental import pallas as pl
from jax.experimental.pallas import tpu as pltpu

C = 16
CL = 8


def kernel(x):
    m_per, n = x.shape
    half = m_per // 2
    rows = half // C
    lrows = m_per // CL

    def body(x_ref, out_ref, vbuf, in_sems, out_sems, sx, rx, sy, ry):
        mx = lax.axis_index("x")
        my = lax.axis_index("y")
        mz = lax.axis_index("z")
        partner_x = (1 - mx, my, mz)
        partner_y = (mx, 1 - my, mz)

        x_rdmas = []
        for i in range(C):
            src_off = my * half + i * rows
            dst_off = mx * m_per + my * half + i * rows
            r = pltpu.make_async_remote_copy(
                src_ref=x_ref.at[pl.ds(src_off, rows), :],
                dst_ref=out_ref.at[pl.ds(dst_off, rows), :],
                send_sem=sx.at[i],
                recv_sem=rx.at[i],
                device_id=partner_x,
                device_id_type=pl.DeviceIdType.MESH,
            )
            r.start()
            x_rdmas.append(r)

        in_cps = []
        for i in range(CL):
            s = i % 2
            c = pltpu.make_async_copy(
                x_ref.at[pl.ds(i * lrows, lrows), :], vbuf.at[s], in_sems.at[s]
            )
            in_cps.append(c)
        in_cps[0].start()
        in_cps[1].start()
        tail_outs = []
        for i in range(CL):
            s = i % 2
            in_cps[i].wait()
            o = pltpu.make_async_copy(
                vbuf.at[s],
                out_ref.at[pl.ds(mx * m_per + i * lrows, lrows), :],
                out_sems.at[s],
            )
            o.start()
            if i + 2 < CL:
                o.wait()
                in_cps[i + 2].start()
            else:
                tail_outs.append(o)

        y_rdmas = []
        for i in range(C):
            x_rdmas[i].wait_recv()
            off = (1 - mx) * m_per + my * half + i * rows
            r = pltpu.make_async_remote_copy(
                src_ref=out_ref.at[pl.ds(off, rows), :],
                dst_ref=out_ref.at[pl.ds(off, rows), :],
                send_sem=sy.at[i],
                recv_sem=ry.at[i],
                device_id=partner_y,
                device_id_type=pl.DeviceIdType.MESH,
            )
            r.start()
            y_rdmas.append(r)

        for i in range(C):
            x_rdmas[i].wait_send()
            y_rdmas[i].wait()
        for o in tail_outs:
            o.wait()

    return pl.pallas_call(
        body,
        out_shape=jax.ShapeDtypeStruct((2 * m_per, n), x.dtype),
        in_specs=[pl.BlockSpec(memory_space=pltpu.MemorySpace.HBM)],
        out_specs=pl.BlockSpec(memory_space=pltpu.MemorySpace.HBM),
        scratch_shapes=[
            pltpu.VMEM((2, m_per // CL, n), jnp.float32),
            pltpu.SemaphoreType.DMA((2,)),
            pltpu.SemaphoreType.DMA((2,)),
            pltpu.SemaphoreType.DMA((C,)),
            pltpu.SemaphoreType.DMA((C,)),
            pltpu.SemaphoreType.DMA((C,)),
            pltpu.SemaphoreType.DMA((C,)),
        ],
    )(x)


# device time: 827778 ns/iter; 5.1447x vs baseline; 1.1959x over previous
import jax
import jax.numpy as jnp
from jax import lax
from jax.experimental import pallas as pl
from jax.experimental.pallas import tpu as pltpu

Q = 8192
MCH = 1024
NM = Q // MCH
DX0, DY0, DZ0 = 0, 2728, 5456
SDX = (1024, 1024, 680)
SDY = (1024, 1024, 680)
SDZ = (1024, 1024, 688)

CL = 8


def kernel(x):
    m_per, n = x.shape
    lrows = m_per // CL

    def body(
        x_ref,
        out_ref,
        vbuf,
        in_sems,
        out_sems,
        s_xm, r_xm, s_xd, r_xd,
        s_ym, r_ym, s_yr, r_yr,
        s_zm, r_zm, s_zr, r_zr,
    ):
        mx = lax.axis_index("x")
        my = lax.axis_index("y")
        mz = lax.axis_index("z")
        b = mz % 2
        p_x = (1 - mx, my, mz)
        p_y = (mx, 1 - my, mz)
        p_z = (mx, my, mz + 1 - 2 * b)

        me_off = (my + 2 * b) * Q
        y_off = ((1 - my) + 2 * b) * Q
        z_off = (my + 2 * (1 - b)) * Q
        diag_off = ((1 - my) + 2 * (1 - b)) * Q

        own_base = mx * m_per
        rem_base = (1 - mx) * m_per

        def mk_stream(src_ref, s0, d0, sizes, ssem, rsem, dev):
            rs, off = [], 0
            for k, sz in enumerate(sizes):
                rs.append(
                    pltpu.make_async_remote_copy(
                        src_ref=src_ref.at[pl.ds(s0 + off, sz), :],
                        dst_ref=out_ref.at[pl.ds(d0 + off, sz), :],
                        send_sem=ssem.at[k],
                        recv_sem=rsem.at[k],
                        device_id=dev,
                        device_id_type=pl.DeviceIdType.MESH,
                    )
                )
                off += sz
            return rs

        MAIN = (MCH,) * NM
        x_main = mk_stream(x_ref, me_off, own_base + me_off, MAIN, s_xm, r_xm, p_x)
        x_diag = mk_stream(
            x_ref, diag_off + DX0, own_base + diag_off + DX0, SDX, s_xd, r_xd, p_x
        )
        y_main = mk_stream(
            out_ref, rem_base + me_off, rem_base + me_off, MAIN, s_ym, r_ym, p_y
        )
        y_relay = mk_stream(
            out_ref,
            rem_base + z_off + DY0,
            rem_base + z_off + DY0,
            SDY,
            s_yr, r_yr, p_y,
        )
        z_main = mk_stream(
            out_ref, rem_base + me_off, rem_base + me_off, MAIN, s_zm, r_zm, p_z
        )
        z_relay = mk_stream(
            out_ref,
            rem_base + y_off + DZ0,
            rem_base + y_off + DZ0,
            SDZ,
            s_zr, r_zr, p_z,
        )

        for r in x_main:
            r.start()
        for r in x_diag:
            r.start()

        in_cps = []
        for i in range(CL):
            s = i % 2
            c = pltpu.make_async_copy(
                x_ref.at[pl.ds(i * lrows, lrows), :], vbuf.at[s], in_sems.at[s]
            )
            in_cps.append(c)
        in_cps[0].start()
        in_cps[1].start()
        tail_outs = []
        for i in range(CL):
            s = i % 2
            in_cps[i].wait()
            o = pltpu.make_async_copy(
                vbuf.at[s],
                out_ref.at[pl.ds(own_base + i * lrows, lrows), :],
                out_sems.at[s],
            )
            o.start()
            if i + 2 < CL:
                o.wait()
                in_cps[i + 2].start()
            else:
                tail_outs.append(o)

        z_cur = 0
        for i in range(NM):
            x_main[i].wait_recv()
            y_main[i].start()
            z_main[i].start()
            if 4 <= i <= 6:
                j = i - 4
                while z_cur <= j + 3:
                    z_main[z_cur].wait_recv()
                    z_cur += 1
                y_relay[j].start()

        y_cur = 0
        while y_cur <= 6:
            y_main[y_cur].wait_recv()
            y_cur += 1
        z_relay[0].start()
        y_main[7].wait_recv()
        z_relay[1].start()
        z_relay[2].start()

        while z_cur < NM:
            z_main[z_cur].wait_recv()
            z_cur += 1
        for r in x_diag:
            r.wait_recv()
        for r in y_relay:
            r.wait_recv()
        for r in z_relay:
            r.wait_recv()
        for r in x_main + x_diag + y_main + y_relay + z_main + z_relay:
            r.wait_send()
        for o in tail_outs:
            o.wait()

    return pl.pallas_call(
        body,
        out_shape=jax.ShapeDtypeStruct((2 * m_per, n), x.dtype),
        in_specs=[pl.BlockSpec(memory_space=pltpu.MemorySpace.HBM)],
        out_specs=pl.BlockSpec(memory_space=pltpu.MemorySpace.HBM),
        scratch_shapes=[
            pltpu.VMEM((2, m_per // CL, n), jnp.float32),
            pltpu.SemaphoreType.DMA((2,)),
            pltpu.SemaphoreType.DMA((2,)),
            pltpu.SemaphoreType.DMA((NM,)),
            pltpu.SemaphoreType.DMA((NM,)),
            pltpu.SemaphoreType.DMA((3,)),
            pltpu.SemaphoreType.DMA((3,)),
            pltpu.SemaphoreType.DMA((NM,)),
            pltpu.SemaphoreType.DMA((NM,)),
            pltpu.SemaphoreType.DMA((3,)),
            pltpu.SemaphoreType.DMA((3,)),
            pltpu.SemaphoreType.DMA((NM,)),
            pltpu.SemaphoreType.DMA((NM,)),
            pltpu.SemaphoreType.DMA((3,)),
            pltpu.SemaphoreType.DMA((3,)),
        ],
    )(x)


# device time: 790393 ns/iter; 5.3880x vs baseline; 1.0473x over previous
import jax
import jax.numpy as jnp
from jax import lax
from jax.experimental import pallas as pl
from jax.experimental.pallas import tpu as pltpu

Q = 8192
MCH = 1024
NM = Q // MCH
DX0, DY0, DZ0 = 0, 2728, 5456
SDX = (1024, 1024, 680)
SDY = (1024, 1024, 680)
SDZ = (1024, 1024, 688)

CL = 8


def kernel(x):
    m_per, n = x.shape
    lrows = m_per // CL

    def body(
        x_ref,
        out_ref,
        vbuf,
        in_sems,
        out_sems,
        s_xm, r_xm, s_xd, r_xd,
        s_ym, r_ym, s_yr, r_yr,
        s_zm, r_zm, s_zr, r_zr,
    ):
        mx = lax.axis_index("x")
        my = lax.axis_index("y")
        mz = lax.axis_index("z")
        b = mz % 2
        p_x = (1 - mx, my, mz)
        p_y = (mx, 1 - my, mz)
        p_z = (mx, my, mz + 1 - 2 * b)

        me_off = (my + 2 * b) * Q
        y_off = ((1 - my) + 2 * b) * Q
        z_off = (my + 2 * (1 - b)) * Q
        diag_off = ((1 - my) + 2 * (1 - b)) * Q

        own_base = mx * m_per
        rem_base = (1 - mx) * m_per

        def mk_stream(src_ref, s0, d0, sizes, ssem, rsem, dev):
            rs, off = [], 0
            for k, sz in enumerate(sizes):
                rs.append(
                    pltpu.make_async_remote_copy(
                        src_ref=src_ref.at[pl.ds(s0 + off, sz), :],
                        dst_ref=out_ref.at[pl.ds(d0 + off, sz), :],
                        send_sem=ssem.at[k],
                        recv_sem=rsem.at[k],
                        device_id=dev,
                        device_id_type=pl.DeviceIdType.MESH,
                    )
                )
                off += sz
            return rs

        MAIN = (MCH,) * NM
        x_main = mk_stream(x_ref, me_off, own_base + me_off, MAIN, s_xm, r_xm, p_x)
        x_diag = mk_stream(
            x_ref, diag_off + DX0, own_base + diag_off + DX0, SDX, s_xd, r_xd, p_x
        )
        y_main = mk_stream(
            out_ref, rem_base + me_off, rem_base + me_off, MAIN, s_ym, r_ym, p_y
        )
        y_relay = mk_stream(
            out_ref,
            rem_base + z_off + DY0,
            rem_base + z_off + DY0,
            SDY,
            s_yr, r_yr, p_y,
        )
        z_main = mk_stream(
            out_ref, rem_base + me_off, rem_base + me_off, MAIN, s_zm, r_zm, p_z
        )
        z_relay = mk_stream(
            out_ref,
            rem_base + y_off + DZ0,
            rem_base + y_off + DZ0,
            SDZ,
            s_zr, r_zr, p_z,
        )

        for r in x_main:
            r.start()
        for r in x_diag:
            r.start()

        in_cps = []
        for i in range(CL):
            c = pltpu.make_async_copy(
                x_ref.at[pl.ds(i * lrows, lrows), :],
                vbuf.at[i % 2],
                in_sems.at[i % 2],
            )
            in_cps.append(c)
        in_cps[0].start()
        in_cps[1].start()
        tail_outs = []

        z_cur = 0
        for i in range(NM):
            x_main[i].wait_recv()
            y_main[i].start()
            z_main[i].start()
            if 4 <= i <= 6:
                j = i - 4
                while z_cur <= j + 3:
                    z_main[z_cur].wait_recv()
                    z_cur += 1
                y_relay[j].start()
            s = i % 2
            in_cps[i].wait()
            o = pltpu.make_async_copy(
                vbuf.at[s],
                out_ref.at[pl.ds(own_base + i * lrows, lrows), :],
                out_sems.at[s],
            )
            o.start()
            if i + 2 < CL:
                o.wait()
                in_cps[i + 2].start()
            else:
                tail_outs.append(o)

        y_cur = 0
        while y_cur <= 6:
            y_main[y_cur].wait_recv()
            y_cur += 1
        z_relay[0].start()
        y_main[7].wait_recv()
        z_relay[1].start()
        z_relay[2].start()

        while z_cur < NM:
            z_main[z_cur].wait_recv()
            z_cur += 1
        for r in x_diag:
            r.wait_recv()
        for r in y_relay:
            r.wait_recv()
        for r in z_relay:
            r.wait_recv()
        for r in x_main + x_diag + y_main + y_relay + z_main + z_relay:
            r.wait_send()
        for o in tail_outs:
            o.wait()

    return pl.pallas_call(
        body,
        out_shape=jax.ShapeDtypeStruct((2 * m_per, n), x.dtype),
        in_specs=[pl.BlockSpec(memory_space=pltpu.MemorySpace.HBM)],
        out_specs=pl.BlockSpec(memory_space=pltpu.MemorySpace.HBM),
        scratch_shapes=[
            pltpu.VMEM((2, m_per // CL, n), jnp.float32),
            pltpu.SemaphoreType.DMA((2,)),
            pltpu.SemaphoreType.DMA((2,)),
            pltpu.SemaphoreType.DMA((NM,)),
            pltpu.SemaphoreType.DMA((NM,)),
            pltpu.SemaphoreType.DMA((3,)),
            pltpu.SemaphoreType.DMA((3,)),
            pltpu.SemaphoreType.DMA((NM,)),
            pltpu.SemaphoreType.DMA((NM,)),
            pltpu.SemaphoreType.DMA((3,)),
            pltpu.SemaphoreType.DMA((3,)),
            pltpu.SemaphoreType.DMA((NM,)),
            pltpu.SemaphoreType.DMA((NM,)),
            pltpu.SemaphoreType.DMA((3,)),
            pltpu.SemaphoreType.DMA((3,)),
        ],
    )(x)
